# baseline (speedup 1.0000x reference)
import jax
import jax.numpy as jnp
from jax.experimental import pallas as pl
from jax.experimental.pallas import tpu as pltpu

_ALPHA = 0.1
_RATE = 0.7

_TM1 = 128
_TM2 = 256
_VMEM = 60 * 1024 * 1024


def _leaky(v):
    return jnp.where(v > 0.0, v, _ALPHA * v)


def _dot(a, b):
    return jnp.dot(a, b, preferred_element_type=jnp.float32)


def _fused_body(vu_s_ref, vu_t_ref, uv_s_ref, uv_t_ref,
                xs_ref, xt_ref, w1_ref, b1_ref, w2_ref, b2_ref,
                w3_ref, w4_ref, b3_ref, b4_ref, sf_ref, tf_ref,
                wsc_ref, wsf_ref, wtc_ref, wtf_ref, bu_ref,
                om_ref, ol_ref,
                sup1_s_ref, sup1_t_ref, sup_s_ref, sup_t_ref,
                *, p1, tm1, fdim):
    i = pl.program_id(0)

    @pl.when(i == 0)
    def _():
        sup1_s_ref[...] = _dot(xs_ref[...], w1_ref[...])
        sup1_t_ref[...] = _dot(xt_ref[...], w2_ref[...])

    @pl.when(i < p1)
    def _():
        hs = _leaky(_dot(vu_s_ref[...], sup1_s_ref[...]) + b1_ref[...])
        sup_s_ref[pl.ds(i * tm1, tm1), :] = _dot(hs, w3_ref[...])
        ht = _leaky(_dot(vu_t_ref[...], sup1_t_ref[...]) + b2_ref[...])
        sup_t_ref[pl.ds(i * tm1, tm1), :] = _dot(ht, w4_ref[...])

    @pl.when(i >= p1)
    def _():
        s_cat = _leaky(_dot(uv_s_ref[...], sup_s_ref[...]) + b3_ref[...])
        t_cat = _leaky(_dot(uv_t_ref[...], sup_t_ref[...]) + b4_ref[...])
        out = _dot(s_cat, wsc_ref[...])
        out = out + _dot(sf_ref[...], wsf_ref[...])
        out = out + _dot(t_cat, wtc_ref[...])
        out = out + _dot(tf_ref[...], wtf_ref[...])
        out = out + bu_ref[...]
        om_ref[...] = out[:, :fdim]
        ol_ref[...] = out[:, fdim:]


def kernel(gc1_w, gc1_b, gc2_w, gc2_b,
           gc3_mean_w, gc3_mean_b, gc3_logstd_w, gc3_logstd_b,
           gc4_mean_w, gc4_mean_b, gc4_logstd_w, gc4_logstd_b,
           union_source_mean_w, union_source_mean_b,
           union_source_logstd_w, union_source_logstd_b,
           union_target_mean_w, union_target_mean_b,
           union_target_logstd_w, union_target_logstd_b,
           source_ufea, target_ufea,
           source_UV_adj, source_VU_adj, target_UV_adj, target_VU_adj):
    fdim = source_ufea.shape[1]
    n_user, n_in = source_ufea.shape
    two_f = 2 * fdim
    n_hid = gc1_w.shape[1]
    n_item = source_VU_adj.shape[0]

    w3 = jnp.concatenate([gc3_mean_w, gc3_logstd_w], axis=1)
    b3 = jnp.concatenate([gc3_mean_b, gc3_logstd_b])
    w4 = jnp.concatenate([gc4_mean_w, gc4_logstd_w], axis=1)
    b4 = jnp.concatenate([gc4_mean_b, gc4_logstd_b])

    def _split(w):
        return w[:, :fdim].T, w[:, fdim:].T

    wh_sm, wf_sm = _split(union_source_mean_w)
    wh_sl, wf_sl = _split(union_source_logstd_w)
    wh_tm, wf_tm = _split(union_target_mean_w)
    wh_tl, wf_tl = _split(union_target_logstd_w)

    zeros = jnp.zeros((fdim, fdim), jnp.float32)
    rate = jnp.float32(_RATE)
    w_sc = jnp.block([[wh_sm, zeros], [zeros, wh_sl]]) * rate
    w_tc = jnp.block([[wh_tm, zeros], [zeros, wh_tl]]) * (1.0 - rate)
    w_sf = jnp.concatenate([wf_sm, wf_sl], axis=1) * rate
    w_tf = jnp.concatenate([wf_tm, wf_tl], axis=1) * (1.0 - rate)
    b_u = (rate * jnp.concatenate([union_source_mean_b, union_source_logstd_b])
           + (1.0 - rate) * jnp.concatenate([union_target_mean_b,
                                             union_target_logstd_b]))

    tm1 = min(_TM1, n_item)
    tm2 = min(_TM2, n_user)
    p1 = n_item // tm1
    p2 = n_user // tm2

    vu = lambda i: (jnp.minimum(i, p1 - 1), 0)
    uv = lambda i: (jnp.maximum(i - p1, 0), 0)
    pin = lambda i: (0, 0)

    import functools
    mean, logstd = pl.pallas_call(
        functools.partial(_fused_body, p1=p1, tm1=tm1, fdim=fdim),
        grid=(p1 + p2,),
        in_specs=[
            pl.BlockSpec((tm1, n_user), vu),
            pl.BlockSpec((tm1, n_user), vu),
            pl.BlockSpec((tm2, n_item), uv),
            pl.BlockSpec((tm2, n_item), uv),
            pl.BlockSpec((n_user, n_in), pin),
            pl.BlockSpec((n_user, n_in), pin),
            pl.BlockSpec((n_in, n_hid), pin),
            pl.BlockSpec((1, n_hid), pin),
            pl.BlockSpec((n_in, n_hid), pin),
            pl.BlockSpec((1, n_hid), pin),
            pl.BlockSpec((n_hid, two_f), pin),
            pl.BlockSpec((n_hid, two_f), pin),
            pl.BlockSpec((1, two_f), pin),
            pl.BlockSpec((1, two_f), pin),
            pl.BlockSpec((tm2, fdim), uv),
            pl.BlockSpec((tm2, fdim), uv),
            pl.BlockSpec((two_f, two_f), pin),
            pl.BlockSpec((fdim, two_f), pin),
            pl.BlockSpec((two_f, two_f), pin),
            pl.BlockSpec((fdim, two_f), pin),
            pl.BlockSpec((1, two_f), pin),
        ],
        out_specs=[
            pl.BlockSpec((tm2, fdim), uv),
            pl.BlockSpec((tm2, fdim), uv),
        ],
        out_shape=[
            jax.ShapeDtypeStruct((n_user, fdim), jnp.float32),
            jax.ShapeDtypeStruct((n_user, fdim), jnp.float32),
        ],
        scratch_shapes=[
            pltpu.VMEM((n_user, n_hid), jnp.float32),
            pltpu.VMEM((n_user, n_hid), jnp.float32),
            pltpu.VMEM((n_item, two_f), jnp.float32),
            pltpu.VMEM((n_item, two_f), jnp.float32),
        ],
        compiler_params=pltpu.CompilerParams(
            dimension_semantics=("arbitrary",),
            vmem_limit_bytes=_VMEM,
        ),
    )(source_VU_adj, target_VU_adj, source_UV_adj, target_UV_adj,
      source_ufea, target_ufea,
      gc1_w, gc1_b.reshape(1, -1), gc2_w, gc2_b.reshape(1, -1),
      w3, w4, b3.reshape(1, -1), b4.reshape(1, -1),
      source_ufea, target_ufea,
      w_sc, w_sf, w_tc, w_tf, b_u.reshape(1, -1))

    return mean, logstd
